# baseline (speedup 1.0000x reference)
import functools

import jax
import jax.numpy as jnp
from jax.experimental import pallas as pl
from jax.experimental.pallas import tpu as pltpu

DC = 8


def _body(NC, S, Dm, q_ref, k_ref, v_ref, wq_ref, wk_ref, wv_ref, wo_ref,
          bv_ref, bo_ref, bq_s, bk_s, ws_s, out_ref, attn_ref,
          qt3, kts, vps, acc):
    f32 = jnp.float32
    c = pl.program_id(1)

    @pl.when(c == 0)
    def _proj():
        kts[...] = jax.lax.dot_general(
            wk_ref[...], k_ref[0], (((1,), (1,)), ((), ())),
            preferred_element_type=f32)
        vps[...] = jax.lax.dot_general(
            v_ref[0], wv_ref[...], (((1,), (1,)), ((), ())),
            preferred_element_type=f32) + bv_ref[...]
        for i in range(NC):
            wq_rows = wq_ref[i * DC:(i + 1) * DC, :]
            qt3[i] = jax.lax.dot_general(
                wq_rows, q_ref[0], (((1,), (1,)), ((), ())),
                preferred_element_type=f32)
        acc[...] = jnp.zeros((S, S), f32)

    qtc = qt3[c]
    base = pl.multiple_of(c * DC, DC)
    ktc = kts[pl.ds(base, DC), :]
    for r in range(DC):
        d = c * DC + r
        w_r = ws_s[0, d]
        b_r = bq_s[0, d] + bk_s[0, d]
        krow = ktc[r:r + 1, :] + b_r
        qcol = jnp.transpose(qtc[r:r + 1, :], (1, 0))
        acc[...] = acc[...] + w_r * jnp.tanh(qcol + krow)

    @pl.when(c == NC - 1)
    def _epi():
        sc = acc[...]
        m = jnp.max(sc, axis=1, keepdims=True)
        e = jnp.exp(sc - m)
        s = jnp.sum(e, axis=1, keepdims=True)
        p = e / s
        attn_ref[0, 0] = p
        av = jax.lax.dot_general(p, vps[...], (((1,), (0,)), ((), ())),
                                 preferred_element_type=f32)
        out_ref[0] = jax.lax.dot_general(
            av, wo_ref[...], (((1,), (1,)), ((), ())),
            preferred_element_type=f32) + bo_ref[...]


def _fwd(query, key, value, Wq, bq, Wk, bk, Wv, bv, Ws, bs, Wo, bo,
         interpret=False):
    f32 = jnp.float32
    B, S, Dm = query.shape
    NC = Dm // DC
    body = functools.partial(_body, NC, S, Dm)

    in_specs = [
        pl.BlockSpec((1, S, Dm), lambda b, c: (b, 0, 0)),
        pl.BlockSpec((1, S, Dm), lambda b, c: (b, 0, 0)),
        pl.BlockSpec((1, S, Dm), lambda b, c: (b, 0, 0)),
        pl.BlockSpec((Dm, Dm), lambda b, c: (0, 0)),
        pl.BlockSpec((Dm, Dm), lambda b, c: (0, 0)),
        pl.BlockSpec((Dm, Dm), lambda b, c: (0, 0)),
        pl.BlockSpec((Dm, Dm), lambda b, c: (0, 0)),
        pl.BlockSpec((1, Dm), lambda b, c: (0, 0)),
        pl.BlockSpec((1, Dm), lambda b, c: (0, 0)),
        pl.BlockSpec(memory_space=pltpu.SMEM),
        pl.BlockSpec(memory_space=pltpu.SMEM),
        pl.BlockSpec(memory_space=pltpu.SMEM),
    ]
    out_specs = [
        pl.BlockSpec((1, S, Dm), lambda b, c: (b, 0, 0)),
        pl.BlockSpec((1, 1, S, S), lambda b, c: (b, 0, 0, 0)),
    ]
    out_shape = [
        jax.ShapeDtypeStruct((B, S, Dm), f32),
        jax.ShapeDtypeStruct((B, 1, S, S), f32),
    ]
    scratch = [
        pltpu.VMEM((NC, DC, S), f32),
        pltpu.VMEM((Dm, S), f32),
        pltpu.VMEM((S, Dm), f32),
        pltpu.VMEM((S, S), f32),
    ]
    out, attn = pl.pallas_call(
        body,
        grid=(B, NC),
        in_specs=in_specs,
        out_specs=out_specs,
        out_shape=out_shape,
        scratch_shapes=scratch,
        compiler_params=pltpu.CompilerParams(
            dimension_semantics=("parallel", "arbitrary"),
            vmem_limit_bytes=48 * 1024 * 1024,
        ),
        interpret=interpret,
    )(query, key, value, Wq, Wk, Wv, Wo,
      bv.reshape(1, Dm), bo.reshape(1, Dm),
      bq.reshape(1, Dm), bk.reshape(1, Dm), Ws.reshape(1, Dm))
    return out, attn


def kernel(query, key, value, Wq, bq, Wk, bk, Wv, bv, Ws, bs, Wo, bo):
    return _fwd(query, key, value, Wq, bq, Wk, bk, Wv, bv, Ws, bs, Wo, bo)
